# baseline (speedup 1.0000x reference)
import functools

import jax
import jax.numpy as jnp
from jax import lax
from jax.experimental import pallas as pl
from jax.experimental.pallas import tpu as pltpu
from jax.experimental.pallas import tpu_sc as plsc


_NC = 2
_NS = 16
_NW = _NC * _NS
_C = 4
_NBUF = 4
_L = 16

_SPLIT = 1024
_TC_BS = 256


def _sc_body(row0, sc_rows, seq_len, batch, d_model,
             x_hbm, pe_hbm, out_hbm, xv, pev, *sems):
    rows_per_w = sc_rows // _NW
    n_chunks = rows_per_w // _C
    n_groups = n_chunks // _NBUF
    n_vec = d_model // _L
    sem_ix = sems[0:_NBUF]
    sem_ip = sems[_NBUF:2 * _NBUF]
    sem_o = sems[2 * _NBUF:3 * _NBUF]
    wid = lax.axis_index("s") * _NC + lax.axis_index("c")
    base = row0 + wid * rows_per_w

    def in_copies(g, b):
        row = base + g * _C
        return (
            pltpu.make_async_copy(x_hbm.at[pl.ds(row, _C)], xv.at[b],
                                  sem_ix[b]),
            pltpu.make_async_copy(pe_hbm.at[pl.ds(row, _C)], pev.at[b],
                                  sem_ip[b]),
        )

    def out_copy(g, b):
        row = base + g * _C
        return pltpu.make_async_copy(xv.at[b], out_hbm.at[pl.ds(row, _C)],
                                     sem_o[b])

    def start_in(g, b):
        cx, cp = in_copies(g, b)
        cx.start()
        cp.start()

    def compute(b):
        @plsc.parallel_loop(0, _C * n_vec, unroll=4)
        def _(t):
            s = t // n_vec
            off = (t % n_vec) * _L
            pv = pev[b, s, pl.ds(off, _L)]
            for bb in range(batch):
                xv[b, s, bb, pl.ds(off, _L)] = xv[b, s, bb, pl.ds(off, _L)] + pv

    for b in range(_NBUF):
        start_in(b, b)

    def group(k, carry):
        for b in range(_NBUF):
            g = k * _NBUF + b
            cx, cp = in_copies(g, b)
            cx.wait()
            cp.wait()
            compute(b)
            out_copy(g, b).start()
            bp = (b + 2) % _NBUF

            @pl.when(jnp.logical_and(g - 2 >= 0, g + 2 < n_chunks))
            def _():
                out_copy(g - 2, bp).wait()
                start_in(g + 2, bp)
        return carry

    lax.fori_loop(0, n_groups, group, 0, unroll=False)

    for b in range(_NBUF):
        g = (n_groups - 1) * _NBUF + b
        out_copy(g, b).wait()


def _sc_add(x, pe, row0):
    seq_len, batch, d_model = x.shape
    mesh = plsc.VectorSubcoreMesh(
        core_axis_name="c", subcore_axis_name="s",
        num_cores=_NC, num_subcores=_NS,
    )
    body = functools.partial(_sc_body, row0, seq_len - row0, seq_len, batch,
                             d_model)
    return pl.kernel(
        body,
        out_type=jax.ShapeDtypeStruct((seq_len, batch, d_model), x.dtype),
        mesh=mesh,
        scratch_types=[
            pltpu.VMEM((_NBUF, _C, batch, d_model), jnp.float32),
            pltpu.VMEM((_NBUF, _C, d_model), jnp.float32),
        ] + [pltpu.SemaphoreType.DMA] * (3 * _NBUF),
    )(x, pe)


def _tc_body(x_ref, pe_ref, acc_ref, o_ref):
    del acc_ref
    o_ref[...] = x_ref[...] + pe_ref[...][:, None, :]


def _tc_add_into(x, pe, acc, m):
    seq_len, batch, d_model = x.shape
    return pl.pallas_call(
        _tc_body,
        grid=(m // _TC_BS,),
        in_specs=[
            pl.BlockSpec((_TC_BS, batch, d_model), lambda i: (i, 0, 0)),
            pl.BlockSpec((_TC_BS, d_model), lambda i: (i, 0)),
            pl.BlockSpec(memory_space=pl.ANY),
        ],
        out_specs=pl.BlockSpec((_TC_BS, batch, d_model), lambda i: (i, 0, 0)),
        out_shape=jax.ShapeDtypeStruct((seq_len, batch, d_model), x.dtype),
        input_output_aliases={2: 0},
    )(x, pe, acc)


def kernel(x, pe):
    seq_len, batch, d_model = x.shape
    pe = pe[:seq_len]
    sc_full = _sc_add(x, pe, _SPLIT)
    return _tc_add_into(x, pe, sc_full, _SPLIT)


# speedup vs baseline: 1.0346x; 1.0346x over previous
import functools

import jax
import jax.numpy as jnp
from jax import lax
from jax.experimental import pallas as pl
from jax.experimental.pallas import tpu as pltpu
from jax.experimental.pallas import tpu_sc as plsc


_NC = 2
_NS = 16
_NW = _NC * _NS
_C = 4
_NBUF = 4
_L = 16


def _sc_body(seq_len, batch, d_model, x_hbm, pe_hbm, out_hbm, xv, pev, *sems):
    rows_per_w = seq_len // _NW
    n_chunks = rows_per_w // _C
    n_groups = n_chunks // _NBUF
    n_vec = d_model // _L
    sem_ix = sems[0:_NBUF]
    sem_ip = sems[_NBUF:2 * _NBUF]
    sem_o = sems[2 * _NBUF:3 * _NBUF]
    wid = lax.axis_index("s") * _NC + lax.axis_index("c")
    base = wid * rows_per_w

    def in_copies(g, b):
        row = base + g * _C
        return (
            pltpu.make_async_copy(x_hbm.at[pl.ds(row, _C)], xv.at[b],
                                  sem_ix[b]),
            pltpu.make_async_copy(pe_hbm.at[pl.ds(row, _C)], pev.at[b],
                                  sem_ip[b]),
        )

    def out_copy(g, b):
        row = base + g * _C
        return pltpu.make_async_copy(xv.at[b], out_hbm.at[pl.ds(row, _C)],
                                     sem_o[b])

    def start_in(g, b):
        cx, cp = in_copies(g, b)
        cx.start()
        cp.start()

    def compute(b):
        @plsc.parallel_loop(0, _C * n_vec, unroll=4)
        def _(t):
            s = t // n_vec
            off = (t % n_vec) * _L
            pv = pev[b, s, pl.ds(off, _L)]
            for bb in range(batch):
                xv[b, s, bb, pl.ds(off, _L)] = xv[b, s, bb, pl.ds(off, _L)] + pv

    for b in range(_NBUF):
        start_in(b, b)

    def group(k, carry):
        for b in range(_NBUF):
            g = k * _NBUF + b
            cx, cp = in_copies(g, b)
            cx.wait()
            cp.wait()
            compute(b)
            out_copy(g, b).start()
            bp = (b + 2) % _NBUF

            @pl.when(jnp.logical_and(g - 2 >= 0, g + 2 < n_chunks))
            def _():
                out_copy(g - 2, bp).wait()
                start_in(g + 2, bp)
        return carry

    lax.fori_loop(0, n_groups, group, 0, unroll=False)

    for b in range(_NBUF):
        g = (n_groups - 1) * _NBUF + b
        out_copy(g, b).wait()


def kernel(x, pe):
    seq_len, batch, d_model = x.shape
    mesh = plsc.VectorSubcoreMesh(
        core_axis_name="c", subcore_axis_name="s",
        num_cores=_NC, num_subcores=_NS,
    )
    body = functools.partial(_sc_body, seq_len, batch, d_model)
    return pl.kernel(
        body,
        out_type=jax.ShapeDtypeStruct((seq_len, batch, d_model), x.dtype),
        mesh=mesh,
        scratch_types=[
            pltpu.VMEM((_NBUF, _C, batch, d_model), jnp.float32),
            pltpu.VMEM((_NBUF, _C, d_model), jnp.float32),
        ] + [pltpu.SemaphoreType.DMA] * (3 * _NBUF),
    )(x, pe[:seq_len])
